# baseline (speedup 1.0000x reference)
import functools

import jax
import jax.numpy as jnp
from jax import lax
from jax.experimental import pallas as pl
from jax.experimental.pallas import tpu as pltpu
from jax.experimental.pallas import tpu_sc as plsc

NUM_LOCATIONS = 100000
NUM_USERS = 100000
D_MODEL = 128
BATCH = 1024
TIME_SLOTS = 8

_INFO = plsc.get_sparse_core_info()
_NC = _INFO.num_cores
_NS = _INFO.num_subcores
_NW = _NC * _NS
_BPW = BATCH // _NW

_D_LOC = D_MODEL // 2
_D_SMALL = D_MODEL // 4

TILE_N = 2048
_GRID_N = (NUM_LOCATIONS + TILE_N - 1) // TILE_N


def _sc_gather_body(loc_idx_hbm, user_idx_hbm,
                    loc_table_hbm, user_table_hbm,
                    loc_out, user_out,
                    idx_v, loc_v, user_v, sem):
    wid = lax.axis_index("s") * _NC + lax.axis_index("c")
    base = wid * _BPW

    pltpu.sync_copy(loc_idx_hbm.at[pl.ds(base, _BPW)], idx_v)
    pltpu.async_copy(loc_table_hbm.at[idx_v], loc_v, sem).wait()
    pltpu.sync_copy(loc_v, loc_out.at[pl.ds(base, _BPW)])

    pltpu.sync_copy(user_idx_hbm.at[pl.ds(base, _BPW)], idx_v)
    pltpu.async_copy(user_table_hbm.at[idx_v], user_v, sem).wait()
    pltpu.sync_copy(user_v, user_out.at[pl.ds(base, _BPW)])


_sc_gather = functools.partial(
    pl.kernel,
    mesh=plsc.VectorSubcoreMesh(core_axis_name="c", subcore_axis_name="s"),
    out_type=(
        jax.ShapeDtypeStruct((BATCH, _D_LOC), jnp.float32),
        jax.ShapeDtypeStruct((BATCH, _D_SMALL), jnp.float32),
    ),
    scratch_types=[
        pltpu.VMEM((_BPW,), jnp.int32),
        pltpu.VMEM((_BPW, _D_LOC), jnp.float32),
        pltpu.VMEM((_BPW, _D_SMALL), jnp.float32),
        pltpu.SemaphoreType.DMA,
    ],
    compiler_params=pltpu.CompilerParams(use_tc_tiling_on_sc=False),
)(_sc_gather_body)


def _tc_matmul_body(loc_ref, time_ref, tt_ref, user_ref, w_ref, b_ref, out_ref):
    ts = jnp.clip(time_ref[...] // 3, 0, 7)
    onehot = (ts == lax.broadcasted_iota(jnp.int32, (BATCH, TIME_SLOTS), 1))
    time_e = lax.dot_general(
        onehot.astype(jnp.float32), tt_ref[...],
        dimension_numbers=(((1,), (0,)), ((), ())),
        preferred_element_type=jnp.float32)
    combined = jnp.concatenate(
        [loc_ref[...], time_e, user_ref[...]], axis=1)
    acc = lax.dot_general(
        combined.astype(jnp.bfloat16), w_ref[...].astype(jnp.bfloat16),
        dimension_numbers=(((1,), (1,)), ((), ())),
        preferred_element_type=jnp.float32)
    out_ref[...] = acc + b_ref[...]


def _tc_matmul(loc_e, time2, time_table, user_e, W, b2):
    return pl.pallas_call(
        _tc_matmul_body,
        grid=(_GRID_N,),
        in_specs=[
            pl.BlockSpec((BATCH, _D_LOC), lambda j: (0, 0)),
            pl.BlockSpec((BATCH, 1), lambda j: (0, 0)),
            pl.BlockSpec((TIME_SLOTS, _D_SMALL), lambda j: (0, 0)),
            pl.BlockSpec((BATCH, _D_SMALL), lambda j: (0, 0)),
            pl.BlockSpec((TILE_N, D_MODEL), lambda j: (j, 0)),
            pl.BlockSpec((1, TILE_N), lambda j: (0, j)),
        ],
        out_specs=pl.BlockSpec((BATCH, TILE_N), lambda j: (0, j)),
        out_shape=jax.ShapeDtypeStruct((BATCH, NUM_LOCATIONS), jnp.float32),
    )(loc_e, time2, time_table, user_e, W, b2)


def kernel(last_location, last_time, user, loc_table, time_table, user_table, W, b):
    loc_e, user_e = _sc_gather(
        last_location.astype(jnp.int32),
        user.astype(jnp.int32),
        loc_table, user_table)
    return _tc_matmul(loc_e, last_time.astype(jnp.int32).reshape(BATCH, 1),
                      time_table, user_e, W, b.reshape(1, NUM_LOCATIONS))


# speedup vs baseline: 1.8860x; 1.8860x over previous
import functools

import jax
import jax.numpy as jnp
from jax import lax
from jax.experimental import pallas as pl
from jax.experimental.pallas import tpu as pltpu
from jax.experimental.pallas import tpu_sc as plsc

NUM_LOCATIONS = 100000
NUM_USERS = 100000
D_MODEL = 128
BATCH = 1024
TIME_SLOTS = 8

_INFO = plsc.get_sparse_core_info()
_NC = _INFO.num_cores
_NS = _INFO.num_subcores
_NW = _NC * _NS
_BPW = BATCH // _NW

_D_LOC = D_MODEL // 2
_D_SMALL = D_MODEL // 4

TILE_N = 2048
_GRID_N = (NUM_LOCATIONS + TILE_N - 1) // TILE_N


def _sc_gather_body(loc_idx_hbm, user_idx_hbm,
                    loc_table_hbm, user_table_hbm,
                    loc_out, user_out,
                    idx_v, loc_v, user_v, sem):
    wid = lax.axis_index("s") * _NC + lax.axis_index("c")
    base = wid * _BPW

    pltpu.sync_copy(loc_idx_hbm.at[pl.ds(base, _BPW)], idx_v)
    pltpu.async_copy(loc_table_hbm.at[idx_v], loc_v, sem).wait()
    pltpu.sync_copy(loc_v, loc_out.at[pl.ds(base, _BPW)])

    pltpu.sync_copy(user_idx_hbm.at[pl.ds(base, _BPW)], idx_v)
    pltpu.async_copy(user_table_hbm.at[idx_v], user_v, sem).wait()
    pltpu.sync_copy(user_v, user_out.at[pl.ds(base, _BPW)])


_sc_gather = functools.partial(
    pl.kernel,
    mesh=plsc.VectorSubcoreMesh(core_axis_name="c", subcore_axis_name="s"),
    out_type=(
        jax.ShapeDtypeStruct((BATCH, _D_LOC), jnp.float32),
        jax.ShapeDtypeStruct((BATCH, _D_SMALL), jnp.float32),
    ),
    scratch_types=[
        pltpu.VMEM((_BPW,), jnp.int32),
        pltpu.VMEM((_BPW, _D_LOC), jnp.float32),
        pltpu.VMEM((_BPW, _D_SMALL), jnp.float32),
        pltpu.SemaphoreType.DMA,
    ],
    compiler_params=pltpu.CompilerParams(use_tc_tiling_on_sc=False),
)(_sc_gather_body)


def _tc_matmul_body(loc_ref, time_ref, tt_ref, user_ref, w_ref, b_ref, out_ref):
    ts = jnp.clip(time_ref[...] // 3, 0, 7)
    onehot = (ts == lax.broadcasted_iota(jnp.int32, (BATCH, TIME_SLOTS), 1))
    time_e = lax.dot_general(
        onehot.astype(jnp.float32), tt_ref[...],
        dimension_numbers=(((1,), (0,)), ((), ())),
        preferred_element_type=jnp.float32)
    combined = jnp.concatenate(
        [loc_ref[...], time_e, user_ref[...]], axis=1)
    acc = lax.dot_general(
        w_ref[...].astype(jnp.bfloat16), combined.astype(jnp.bfloat16),
        dimension_numbers=(((1,), (1,)), ((), ())),
        preferred_element_type=jnp.float32)
    out_ref[...] = acc + b_ref[...]


def _tc_matmul(loc_e, time2, time_table, user_e, W, b2):
    return pl.pallas_call(
        _tc_matmul_body,
        grid=(_GRID_N,),
        in_specs=[
            pl.BlockSpec((BATCH, _D_LOC), lambda j: (0, 0)),
            pl.BlockSpec((BATCH, 1), lambda j: (0, 0)),
            pl.BlockSpec((TIME_SLOTS, _D_SMALL), lambda j: (0, 0)),
            pl.BlockSpec((BATCH, _D_SMALL), lambda j: (0, 0)),
            pl.BlockSpec((TILE_N, D_MODEL), lambda j: (j, 0)),
            pl.BlockSpec((TILE_N, 1), lambda j: (j, 0)),
        ],
        out_specs=pl.BlockSpec((TILE_N, BATCH), lambda j: (j, 0)),
        out_shape=jax.ShapeDtypeStruct((NUM_LOCATIONS, BATCH), jnp.float32),
    )(loc_e, time2, time_table, user_e, W, b2)


def kernel(last_location, last_time, user, loc_table, time_table, user_table, W, b):
    loc_e, user_e = _sc_gather(
        last_location.astype(jnp.int32),
        user.astype(jnp.int32),
        loc_table, user_table)
    logits_t = _tc_matmul(loc_e, last_time.astype(jnp.int32).reshape(BATCH, 1),
                          time_table, user_e, W, b.reshape(NUM_LOCATIONS, 1))
    return logits_t.T
